# Initial kernel scaffold; baseline (speedup 1.0000x reference)
import jax
import jax.numpy as jnp
from jax.experimental import pallas as pl


def kernel(x, edge_index, edge_label_index, W1l, W1r, b1, W2l, W2r, b2, Wm1, bm1, Wm2, bm2):
    raise NotImplementedError("write your pallas kernel here")


# speedup vs baseline: 4.0392x; 4.0392x over previous
import dataclasses
import functools

import jax
import jax.numpy as jnp
from jax import lax
from jax.experimental import pallas as pl
from jax.experimental.pallas import tpu as pltpu
from jax.experimental.pallas import tpu_sc as plsc

N = 10000
E = 320000
EL = 100000
D_IN = 128
D_H = 256
D_OUT = 128
BNS = float(1.0 / (1.0 + 1e-5) ** 0.5)

NC, NS = 2, 16
NW = NC * NS
EPW = E // NW
AC = 80
ANC = EPW // AC
RPT = 624
ZR = 16

ELP = 102400
GPW = ELP // NW
GC = 128
GNC = GPW // GC

_MESH = plsc.VectorSubcoreMesh(core_axis_name="c", subcore_axis_name="s")

_SC_PARAMS = pltpu.CompilerParams()
if "needs_layout_passes" in pltpu.CompilerParams.__dataclass_fields__:
    _SC_PARAMS = dataclasses.replace(_SC_PARAMS, needs_layout_passes=False)


def _make_agg(with_deg):
    out_type = [jax.ShapeDtypeStruct((NC, N, D_IN), jnp.float32)]
    scratch = [
        pltpu.VMEM((AC,), jnp.int32),
        pltpu.VMEM((AC,), jnp.int32),
        pltpu.VMEM((AC, D_IN), jnp.float32),
        pltpu.VMEM((ZR, D_IN), jnp.float32),
        pltpu.VMEM_SHARED((N, D_IN), jnp.float32),
    ]
    if with_deg:
        out_type.append(jax.ShapeDtypeStruct((NW, N), jnp.float32))
        scratch.append(pltpu.VMEM((N,), jnp.float32))

    def body(src_hbm, dst_hbm, feat_hbm, agg_out, *rest):
        if with_deg:
            deg_out, isx, idx, rows, zbuf, sh_agg, degbuf = rest
        else:
            isx, idx, rows, zbuf, sh_agg = rest
        cid = lax.axis_index("c")
        sid = lax.axis_index("s")
        wid = sid * NC + cid

        zero16 = jnp.zeros((16,), jnp.float32)

        @pl.loop(0, ZR)
        def _(r):
            @pl.loop(0, D_IN, step=16)
            def _(k):
                zbuf[r, pl.ds(k, 16)] = zero16

        if with_deg:
            @pl.loop(0, N, step=16)
            def _(r):
                degbuf[pl.ds(r, 16)] = zero16

        rbase = sid * RPT

        @pl.loop(0, RPT // ZR)
        def _(b):
            pltpu.sync_copy(zbuf, sh_agg.at[pl.ds(rbase + b * ZR, ZR)])

        @pl.when(sid == 0)
        def _():
            pltpu.sync_copy(zbuf, sh_agg.at[pl.ds(NS * RPT, 16)])

        plsc.subcore_barrier()

        ebase = wid * EPW
        one16 = jnp.full((16,), 1.0, jnp.float32)

        @pl.loop(0, ANC)
        def _(ci):
            b0 = ebase + ci * AC
            pltpu.sync_copy(src_hbm.at[pl.ds(b0, AC)], isx)
            pltpu.sync_copy(dst_hbm.at[pl.ds(b0, AC)], idx)
            pltpu.sync_copy(feat_hbm.at[isx], rows)
            pltpu.sync_copy(rows, sh_agg.at[idx], add=True)
            if with_deg:
                for e in range(0, AC, 16):
                    plsc.addupdate_scatter(degbuf, [idx[pl.ds(e, 16)]], one16)

        plsc.subcore_barrier()

        pltpu.sync_copy(sh_agg.at[pl.ds(rbase, RPT)],
                        agg_out.at[cid, pl.ds(rbase, RPT)])
        if with_deg:
            pltpu.sync_copy(degbuf, deg_out.at[wid])

        @pl.when(sid == 0)
        def _():
            pltpu.sync_copy(sh_agg.at[pl.ds(NS * RPT, 16)],
                            agg_out.at[cid, pl.ds(NS * RPT, 16)])

    return pl.kernel(body, out_type=out_type, mesh=_MESH,
                     scratch_types=scratch, compiler_params=_SC_PARAMS)


_agg_deg = _make_agg(True)
_agg = _make_agg(False)


@functools.partial(
    pl.kernel,
    out_type=[jax.ShapeDtypeStruct((ELP, D_OUT), jnp.float32),
              jax.ShapeDtypeStruct((ELP, D_OUT), jnp.float32)],
    mesh=_MESH,
    scratch_types=[
        pltpu.VMEM((GC,), jnp.int32),
        pltpu.VMEM((GC,), jnp.int32),
        pltpu.VMEM((GC, D_OUT), jnp.float32),
        pltpu.VMEM((GC, D_OUT), jnp.float32),
    ],
)
def _decode_gather(z_hbm, e0_hbm, e1_hbm, zs_out, zd_out, i0, i1, r0, r1):
    cid = lax.axis_index("c")
    sid = lax.axis_index("s")
    wid = sid * NC + cid
    gbase = wid * GPW

    @pl.loop(0, GNC)
    def _(ci):
        b0 = gbase + ci * GC
        pltpu.sync_copy(e0_hbm.at[pl.ds(b0, GC)], i0)
        pltpu.sync_copy(e1_hbm.at[pl.ds(b0, GC)], i1)
        pltpu.sync_copy(z_hbm.at[i0], r0)
        pltpu.sync_copy(r0, zs_out.at[pl.ds(b0, GC)])
        pltpu.sync_copy(z_hbm.at[i1], r1)
        pltpu.sync_copy(r1, zd_out.at[pl.ds(b0, GC)])


_RA = 1000


def _dinv_of(degp_ref):
    deg = jnp.sum(degp_ref[...], axis=1)
    return jnp.where(deg > 0, 1.0 / deg, 0.0)


def _stage_a_body(aggp_ref, degp_ref, x_ref, w1l_ref, w1r_ref, b1_ref,
                  w2l_ref, h_ref, hw_ref):
    agg = aggp_ref[0] + aggp_ref[1]
    a = agg * _dinv_of(degp_ref)[:, None]
    pre = (jnp.dot(a, w1l_ref[...], preferred_element_type=jnp.float32)
           + jnp.dot(x_ref[...], w1r_ref[...], preferred_element_type=jnp.float32)
           + b1_ref[...])
    h = jnp.maximum(pre * BNS, 0.0)
    h_ref[...] = h
    hw_ref[...] = jnp.dot(h, w2l_ref[...], preferred_element_type=jnp.float32)


_stage_a = pl.pallas_call(
    _stage_a_body,
    grid=(N // _RA,),
    in_specs=[
        pl.BlockSpec((NC, _RA, D_IN), lambda i: (0, i, 0)),
        pl.BlockSpec((_RA, NW), lambda i: (i, 0)),
        pl.BlockSpec((_RA, D_IN), lambda i: (i, 0)),
        pl.BlockSpec((D_IN, D_H), lambda i: (0, 0)),
        pl.BlockSpec((D_IN, D_H), lambda i: (0, 0)),
        pl.BlockSpec((1, D_H), lambda i: (0, 0)),
        pl.BlockSpec((D_H, D_OUT), lambda i: (0, 0)),
    ],
    out_specs=[
        pl.BlockSpec((_RA, D_H), lambda i: (i, 0)),
        pl.BlockSpec((_RA, D_OUT), lambda i: (i, 0)),
    ],
    out_shape=[
        jax.ShapeDtypeStruct((N, D_H), jnp.float32),
        jax.ShapeDtypeStruct((N, D_OUT), jnp.float32),
    ],
)


def _stage_b_body(aggp_ref, degp_ref, h_ref, w2r_ref, b2_ref, z_ref):
    agg = aggp_ref[0] + aggp_ref[1]
    a = agg * _dinv_of(degp_ref)[:, None]
    pre = (a + jnp.dot(h_ref[...], w2r_ref[...],
                       preferred_element_type=jnp.float32)
           + b2_ref[...])
    z_ref[...] = pre * BNS


_stage_b = pl.pallas_call(
    _stage_b_body,
    grid=(N // _RA,),
    in_specs=[
        pl.BlockSpec((NC, _RA, D_OUT), lambda i: (0, i, 0)),
        pl.BlockSpec((_RA, NW), lambda i: (i, 0)),
        pl.BlockSpec((_RA, D_H), lambda i: (i, 0)),
        pl.BlockSpec((D_H, D_OUT), lambda i: (0, 0)),
        pl.BlockSpec((1, D_OUT), lambda i: (0, 0)),
    ],
    out_specs=pl.BlockSpec((_RA, D_OUT), lambda i: (i, 0)),
    out_shape=jax.ShapeDtypeStruct((N, D_OUT), jnp.float32),
)


_BC = 1024


def _stage_c_body(zs_ref, zd_ref, wa_ref, wb_ref, bm1_ref, wm2_ref, bm2_ref,
                  o_ref):
    hd = (jnp.dot(zs_ref[...], wa_ref[...], preferred_element_type=jnp.float32)
          + jnp.dot(zd_ref[...], wb_ref[...], preferred_element_type=jnp.float32)
          + bm1_ref[...])
    hd = jnp.maximum(hd, 0.0)
    o_ref[...] = (jnp.dot(hd, wm2_ref[...], preferred_element_type=jnp.float32)
                  + bm2_ref[...])


_stage_c = pl.pallas_call(
    _stage_c_body,
    grid=(ELP // _BC,),
    in_specs=[
        pl.BlockSpec((_BC, D_OUT), lambda i: (i, 0)),
        pl.BlockSpec((_BC, D_OUT), lambda i: (i, 0)),
        pl.BlockSpec((D_OUT, D_H), lambda i: (0, 0)),
        pl.BlockSpec((D_OUT, D_H), lambda i: (0, 0)),
        pl.BlockSpec((1, D_H), lambda i: (0, 0)),
        pl.BlockSpec((D_H, 1), lambda i: (0, 0)),
        pl.BlockSpec((1, 1), lambda i: (0, 0)),
    ],
    out_specs=pl.BlockSpec((_BC, 1), lambda i: (i, 0)),
    out_shape=jax.ShapeDtypeStruct((ELP, 1), jnp.float32),
)


def kernel(x, edge_index, edge_label_index, W1l, W1r, b1, W2l, W2r, b2,
           Wm1, bm1, Wm2, bm2):
    src = edge_index[0]
    dst = edge_index[1]

    aggp, degp = _agg_deg(src, dst, x)
    degt = degp.T
    h, hW = _stage_a(aggp, degt, x, W1l, W1r, b1.reshape(1, -1), W2l)
    (agg2p,) = _agg(src, dst, hW)
    z = _stage_b(agg2p, degt, h, W2r, b2.reshape(1, -1))

    pad = jnp.zeros((ELP - EL,), jnp.int32)
    e0 = jnp.concatenate([edge_label_index[0], pad])
    e1 = jnp.concatenate([edge_label_index[1], pad])
    zs, zd = _decode_gather(z, e0, e1)

    out = _stage_c(zs, zd, Wm1[:D_OUT], Wm1[D_OUT:], bm1.reshape(1, -1),
                   Wm2, bm2.reshape(1, 1))
    return out[:EL].reshape(-1)


# speedup vs baseline: 5.7542x; 1.4246x over previous
import dataclasses
import functools

import jax
import jax.numpy as jnp
from jax import lax
from jax.experimental import pallas as pl
from jax.experimental.pallas import tpu as pltpu
from jax.experimental.pallas import tpu_sc as plsc

N = 10000
E = 320000
EL = 100000
D_IN = 128
D_H = 256
D_OUT = 128
BNS = float(1.0 / (1.0 + 1e-5) ** 0.5)

NC, NS = 2, 16
NW = NC * NS
EPW = E // NW
AC = 80
ANC = EPW // AC
RPT = 624
ZR = 16

ELP = 102400
GPW = ELP // NW
GC = 128
GNC = GPW // GC

_MESH = plsc.VectorSubcoreMesh(core_axis_name="c", subcore_axis_name="s")

_SC_PARAMS = pltpu.CompilerParams()
if "needs_layout_passes" in pltpu.CompilerParams.__dataclass_fields__:
    _SC_PARAMS = dataclasses.replace(_SC_PARAMS, needs_layout_passes=False)


def _make_agg(with_deg):
    out_type = [jax.ShapeDtypeStruct((NC, N, D_IN), jnp.float32)]
    scratch = (
        [pltpu.VMEM((AC,), jnp.int32) for _ in range(4)]
        + [pltpu.VMEM((AC,), jnp.int32) for _ in range(4)]
        + [pltpu.VMEM((AC, D_IN), jnp.float32) for _ in range(2)]
        + [
            pltpu.VMEM((ZR, D_IN), jnp.float32),
            pltpu.VMEM_SHARED((N, D_IN), jnp.float32),
        ]
        + [pltpu.SemaphoreType.DMA for _ in range(9)]
    )
    if with_deg:
        out_type.append(jax.ShapeDtypeStruct((NW, N), jnp.float32))
        scratch.append(pltpu.VMEM((N,), jnp.float32))

    def body(src_hbm, dst_hbm, feat_hbm, agg_out, *rest):
        if with_deg:
            deg_out = rest[0]
            rest = rest[1:]
        isx = rest[0:4]
        idxd = rest[4:8]
        rows = rest[8:10]
        zbuf, sh_agg = rest[10:12]
        semi = rest[12:16]
        semg = rest[16:18]
        sems = rest[18:20]
        semz = rest[20]
        degbuf = rest[21] if with_deg else None
        cid = lax.axis_index("c")
        sid = lax.axis_index("s")
        wid = sid * NC + cid

        zero16 = jnp.zeros((16,), jnp.float32)
        one16 = jnp.full((16,), 1.0, jnp.float32)

        @pl.loop(0, ZR)
        def _(r):
            @pl.loop(0, D_IN, step=16)
            def _(k):
                zbuf[r, pl.ds(k, 16)] = zero16

        if with_deg:
            @pl.loop(0, N, step=16)
            def _(r):
                degbuf[pl.ds(r, 16)] = zero16

        rbase = sid * RPT
        nz = RPT // ZR

        @pl.loop(0, nz)
        def _(b):
            pltpu.async_copy(zbuf, sh_agg.at[pl.ds(rbase + b * ZR, ZR)], semz)

        @pl.when(sid == 0)
        def _():
            pltpu.async_copy(zbuf, sh_agg.at[pl.ds(NS * RPT, 16)], semz)

        @pl.loop(0, nz)
        def _(b):
            pltpu.make_async_copy(zbuf, sh_agg.at[pl.ds(0, ZR)], semz).wait()

        @pl.when(sid == 0)
        def _():
            pltpu.make_async_copy(zbuf, sh_agg.at[pl.ds(0, 16)], semz).wait()

        plsc.subcore_barrier()

        ebase = wid * EPW

        def fill(j, c):
            b0 = ebase + c * AC
            pltpu.async_copy(src_hbm.at[pl.ds(b0, AC)], isx[j], semi[j])
            pltpu.async_copy(dst_hbm.at[pl.ds(b0, AC)], idxd[j], semi[j])

        def wait_fill(j):
            pltpu.make_async_copy(src_hbm.at[pl.ds(0, AC)], isx[j],
                                  semi[j]).wait()
            pltpu.make_async_copy(dst_hbm.at[pl.ds(0, AC)], idxd[j],
                                  semi[j]).wait()

        def wait_scatter(b, j):
            pltpu.make_async_copy(rows[b], sh_agg.at[idxd[j]], sems[b]).wait()

        def phase(c, j, b, refill, wait_sc):
            wait_fill(j)
            if wait_sc:
                wait_scatter(b, (j + 2) % 4)
            if refill:
                fill((j + 2) % 4, c + 2)
            pltpu.async_copy(feat_hbm.at[isx[j]], rows[b], semg[b]).wait()
            if with_deg:
                for e in range(0, AC, 16):
                    plsc.addupdate_scatter(degbuf, [idxd[j][pl.ds(e, 16)]],
                                           one16)
            pltpu.async_copy(rows[b], sh_agg.at[idxd[j]], sems[b], add=True)

        fill(0, 0)
        fill(1, 1)
        phase(0, 0, 0, True, False)
        phase(1, 1, 1, True, False)

        @pl.loop(0, (ANC - 5) // 4)
        def _(q):
            c = 4 * q + 2
            phase(c, 2, 0, True, True)
            phase(c + 1, 3, 1, True, True)
            phase(c + 2, 0, 0, True, True)
            phase(c + 3, 1, 1, True, True)

        phase(ANC - 3, 2, 0, True, True)
        phase(ANC - 2, 3, 1, False, True)
        phase(ANC - 1, 0, 0, False, True)
        wait_scatter(1, 3)
        wait_scatter(0, 0)

        plsc.subcore_barrier()

        pltpu.sync_copy(sh_agg.at[pl.ds(rbase, RPT)],
                        agg_out.at[cid, pl.ds(rbase, RPT)])
        if with_deg:
            pltpu.sync_copy(degbuf, deg_out.at[wid])

        @pl.when(sid == 0)
        def _():
            pltpu.sync_copy(sh_agg.at[pl.ds(NS * RPT, 16)],
                            agg_out.at[cid, pl.ds(NS * RPT, 16)])

    return pl.kernel(body, out_type=out_type, mesh=_MESH,
                     scratch_types=scratch, compiler_params=_SC_PARAMS)


_agg_deg = _make_agg(True)
_agg = _make_agg(False)


@functools.partial(
    pl.kernel,
    out_type=[jax.ShapeDtypeStruct((ELP, D_OUT), jnp.float32),
              jax.ShapeDtypeStruct((ELP, D_OUT), jnp.float32)],
    mesh=_MESH,
    scratch_types=(
        [pltpu.VMEM((GC,), jnp.int32) for _ in range(2)]
        + [pltpu.VMEM((GC,), jnp.int32) for _ in range(2)]
        + [pltpu.VMEM((GC, D_OUT), jnp.float32) for _ in range(2)]
        + [pltpu.VMEM((GC, D_OUT), jnp.float32) for _ in range(2)]
        + [pltpu.SemaphoreType.DMA for _ in range(6)]
    ),
    compiler_params=_SC_PARAMS,
)
def _decode_gather(z_hbm, e0_hbm, e1_hbm, zs_out, zd_out, *rest):
    e0b = rest[0:2]
    e1b = rest[2:4]
    ra = rest[4:6]
    rb = rest[6:8]
    semi = rest[8:10]
    semg = rest[10:12]
    semw = rest[12:14]
    cid = lax.axis_index("c")
    sid = lax.axis_index("s")
    wid = sid * NC + cid
    gbase = wid * GPW

    def fill(k, c):
        b0 = gbase + c * GC
        pltpu.async_copy(e0_hbm.at[pl.ds(b0, GC)], e0b[k], semi[k])
        pltpu.async_copy(e1_hbm.at[pl.ds(b0, GC)], e1b[k], semi[k])

    def wait_fill(k):
        pltpu.make_async_copy(e0_hbm.at[pl.ds(0, GC)], e0b[k], semi[k]).wait()
        pltpu.make_async_copy(e1_hbm.at[pl.ds(0, GC)], e1b[k], semi[k]).wait()

    def wait_writes(k):
        pltpu.make_async_copy(ra[k], zs_out.at[pl.ds(0, GC)], semw[k]).wait()
        pltpu.make_async_copy(rb[k], zd_out.at[pl.ds(0, GC)], semw[k]).wait()

    def phase(c, k, refill, wait_w):
        wait_fill(k)
        if wait_w:
            wait_writes(k)
        ga = pltpu.async_copy(z_hbm.at[e0b[k]], ra[k], semg[k])
        gb = pltpu.async_copy(z_hbm.at[e1b[k]], rb[k], semg[k])
        ga.wait()
        gb.wait()
        if refill:
            fill(k, c + 2)
        b0 = gbase + c * GC
        pltpu.async_copy(ra[k], zs_out.at[pl.ds(b0, GC)], semw[k])
        pltpu.async_copy(rb[k], zd_out.at[pl.ds(b0, GC)], semw[k])

    fill(0, 0)
    fill(1, 1)
    phase(0, 0, True, False)
    phase(1, 1, True, False)

    @pl.loop(0, (GNC - 5) // 2)
    def _(q):
        c = 2 * q + 2
        phase(c, 0, True, True)
        phase(c + 1, 1, True, True)

    phase(GNC - 3, 0, True, True)
    phase(GNC - 2, 1, False, True)
    phase(GNC - 1, 0, False, True)
    wait_writes(1)
    wait_writes(0)


_RA = 1000


def _dinv_of(degp_ref):
    deg = jnp.sum(degp_ref[...], axis=1)
    return jnp.where(deg > 0, 1.0 / deg, 0.0)


def _stage_a_body(aggp_ref, degp_ref, x_ref, w1l_ref, w1r_ref, b1_ref,
                  w2l_ref, h_ref, hw_ref):
    agg = aggp_ref[0] + aggp_ref[1]
    a = agg * _dinv_of(degp_ref)[:, None]
    pre = (jnp.dot(a, w1l_ref[...], preferred_element_type=jnp.float32)
           + jnp.dot(x_ref[...], w1r_ref[...], preferred_element_type=jnp.float32)
           + b1_ref[...])
    h = jnp.maximum(pre * BNS, 0.0)
    h_ref[...] = h
    hw_ref[...] = jnp.dot(h, w2l_ref[...], preferred_element_type=jnp.float32)


_stage_a = pl.pallas_call(
    _stage_a_body,
    grid=(N // _RA,),
    in_specs=[
        pl.BlockSpec((NC, _RA, D_IN), lambda i: (0, i, 0)),
        pl.BlockSpec((_RA, NW), lambda i: (i, 0)),
        pl.BlockSpec((_RA, D_IN), lambda i: (i, 0)),
        pl.BlockSpec((D_IN, D_H), lambda i: (0, 0)),
        pl.BlockSpec((D_IN, D_H), lambda i: (0, 0)),
        pl.BlockSpec((1, D_H), lambda i: (0, 0)),
        pl.BlockSpec((D_H, D_OUT), lambda i: (0, 0)),
    ],
    out_specs=[
        pl.BlockSpec((_RA, D_H), lambda i: (i, 0)),
        pl.BlockSpec((_RA, D_OUT), lambda i: (i, 0)),
    ],
    out_shape=[
        jax.ShapeDtypeStruct((N, D_H), jnp.float32),
        jax.ShapeDtypeStruct((N, D_OUT), jnp.float32),
    ],
)


def _stage_b_body(aggp_ref, degp_ref, h_ref, w2r_ref, b2_ref, z_ref):
    agg = aggp_ref[0] + aggp_ref[1]
    a = agg * _dinv_of(degp_ref)[:, None]
    pre = (a + jnp.dot(h_ref[...], w2r_ref[...],
                       preferred_element_type=jnp.float32)
           + b2_ref[...])
    z_ref[...] = pre * BNS


_stage_b = pl.pallas_call(
    _stage_b_body,
    grid=(N // _RA,),
    in_specs=[
        pl.BlockSpec((NC, _RA, D_OUT), lambda i: (0, i, 0)),
        pl.BlockSpec((_RA, NW), lambda i: (i, 0)),
        pl.BlockSpec((_RA, D_H), lambda i: (i, 0)),
        pl.BlockSpec((D_H, D_OUT), lambda i: (0, 0)),
        pl.BlockSpec((1, D_OUT), lambda i: (0, 0)),
    ],
    out_specs=pl.BlockSpec((_RA, D_OUT), lambda i: (i, 0)),
    out_shape=jax.ShapeDtypeStruct((N, D_OUT), jnp.float32),
)


_BC = 1024


def _stage_c_body(zs_ref, zd_ref, wa_ref, wb_ref, bm1_ref, wm2_ref, bm2_ref,
                  o_ref):
    hd = (jnp.dot(zs_ref[...], wa_ref[...], preferred_element_type=jnp.float32)
          + jnp.dot(zd_ref[...], wb_ref[...], preferred_element_type=jnp.float32)
          + bm1_ref[...])
    hd = jnp.maximum(hd, 0.0)
    o_ref[...] = (jnp.dot(hd, wm2_ref[...], preferred_element_type=jnp.float32)
                  + bm2_ref[...])


_stage_c = pl.pallas_call(
    _stage_c_body,
    grid=(ELP // _BC,),
    in_specs=[
        pl.BlockSpec((_BC, D_OUT), lambda i: (i, 0)),
        pl.BlockSpec((_BC, D_OUT), lambda i: (i, 0)),
        pl.BlockSpec((D_OUT, D_H), lambda i: (0, 0)),
        pl.BlockSpec((D_OUT, D_H), lambda i: (0, 0)),
        pl.BlockSpec((1, D_H), lambda i: (0, 0)),
        pl.BlockSpec((D_H, 1), lambda i: (0, 0)),
        pl.BlockSpec((1, 1), lambda i: (0, 0)),
    ],
    out_specs=pl.BlockSpec((_BC, 1), lambda i: (i, 0)),
    out_shape=jax.ShapeDtypeStruct((ELP, 1), jnp.float32),
)


def kernel(x, edge_index, edge_label_index, W1l, W1r, b1, W2l, W2r, b2,
           Wm1, bm1, Wm2, bm2):
    src = edge_index[0]
    dst = edge_index[1]

    aggp, degp = _agg_deg(src, dst, x)
    degt = degp.T
    h, hW = _stage_a(aggp, degt, x, W1l, W1r, b1.reshape(1, -1), W2l)
    (agg2p,) = _agg(src, dst, hW)
    z = _stage_b(agg2p, degt, h, W2r, b2.reshape(1, -1))

    pad = jnp.zeros((ELP - EL,), jnp.int32)
    e0 = jnp.concatenate([edge_label_index[0], pad])
    e1 = jnp.concatenate([edge_label_index[1], pad])
    zs, zd = _decode_gather(z, e0, e1)

    out = _stage_c(zs, zd, Wm1[:D_OUT], Wm1[D_OUT:], bm1.reshape(1, -1),
                   Wm2, bm2.reshape(1, 1))
    return out[:EL].reshape(-1)
